# baseline (speedup 1.0000x reference)
import functools

import jax
import jax.numpy as jnp
from jax import lax
from jax.experimental import pallas as pl
from jax.experimental.pallas import tpu as pltpu
from jax.experimental.pallas import tpu_sc as plsc

_NC = 2
_NS = 16
_NT = _NC * _NS


def _mesh():
    return plsc.VectorSubcoreMesh(
        core_axis_name="c", subcore_axis_name="s",
        num_cores=_NC, num_subcores=_NS)


def _sc_count(row3, ones_b, zb, *, NP, CH, NCHUNK):
    rows_per = NP // _NS
    LAG = 8

    @functools.partial(
        pl.kernel,
        out_type=jax.ShapeDtypeStruct((_NC, NP, 128), jnp.float32),
        mesh=_mesh(),
        scratch_types=[
            pltpu.VMEM((NCHUNK, CH), jnp.int32),
            pltpu.VMEM((CH, 128), jnp.float32),
            pltpu.VMEM_SHARED((NP, 128), jnp.float32),
            pltpu.SemaphoreType.DMA,
        ],
    )
    def k(row_h, ones_h, z_h, out_h, ridx, obuf, acc, ssem):
        cid = lax.axis_index("c")
        sid = lax.axis_index("s")
        tid = cid * _NS + sid
        pltpu.sync_copy(row_h.at[tid], ridx)
        pltpu.sync_copy(ones_h, obuf)
        r0 = sid * rows_per
        pltpu.sync_copy(z_h.at[pl.ds(r0, rows_per)], acc.at[pl.ds(r0, rows_per)])
        plsc.subcore_barrier()

        def body(j, carry):
            pltpu.async_copy(obuf, acc.at[ridx.at[j]], ssem, add=True)

            @pl.when(j >= LAG)
            def _():
                pltpu.make_async_copy(obuf, acc.at[ridx.at[0]], ssem).wait()

            return carry

        lax.fori_loop(0, NCHUNK, body, 0)

        def drain(j, carry):
            pltpu.make_async_copy(obuf, acc.at[ridx.at[0]], ssem).wait()
            return carry

        lax.fori_loop(0, min(LAG, NCHUNK), drain, 0)
        plsc.subcore_barrier()
        pltpu.sync_copy(acc.at[pl.ds(r0, rows_per)],
                        out_h.at[cid, pl.ds(r0, rows_per)])

    return k(row3, ones_b, zb)


def _sc_spmm(y, row_f, col_f, zfull, *, NP, D, CH, NCHUNK):
    rows_per = NP // _NS
    NB = 4
    NG = NCHUNK // NB
    REM = NCHUNK - NG * NB

    @functools.partial(
        pl.kernel,
        out_type=jax.ShapeDtypeStruct((_NC, NP, D), jnp.float32),
        mesh=_mesh(),
        scratch_types=[
            pltpu.VMEM((NB, CH), jnp.int32),
            pltpu.VMEM((NB, CH), jnp.int32),
            *[pltpu.VMEM((CH, D), jnp.float32) for _ in range(NB)],
            pltpu.VMEM_SHARED((NP, D), jnp.float32),
            *[pltpu.SemaphoreType.DMA for _ in range(3 * NB)],
        ],
    )
    def k(y_h, row_h, col_h, z_h, out_h, rbuf, cbuf, b0, b1, b2, b3, acc,
          g0, g1, g2, g3, s0, s1, s2, s3, i0, i1, i2, i3):
        bufs = [b0, b1, b2, b3]
        gsems = [g0, g1, g2, g3]
        ssems = [s0, s1, s2, s3]
        isems = [i0, i1, i2, i3]
        cid = lax.axis_index("c")
        sid = lax.axis_index("s")
        tid = cid * _NS + sid
        r0 = sid * rows_per

        def idx_start(j, b):
            off = tid * NCHUNK * CH + j * CH
            pltpu.async_copy(row_h.at[pl.ds(off, CH)], rbuf.at[b], isems[b])
            pltpu.async_copy(col_h.at[pl.ds(off, CH)], cbuf.at[b], isems[b])

        def idx_wait(b):
            pltpu.make_async_copy(row_h.at[pl.ds(0, CH)], rbuf.at[b], isems[b]).wait()
            pltpu.make_async_copy(row_h.at[pl.ds(0, CH)], cbuf.at[b], isems[b]).wait()

        for b in range(NB):
            idx_start(b, b)
        pltpu.sync_copy(z_h.at[pl.ds(r0, rows_per)], acc.at[pl.ds(r0, rows_per)])
        plsc.subcore_barrier()

        def group(g, carry):
            for b in range(NB):
                idx_wait(b)
                pltpu.async_copy(y_h.at[cbuf.at[b]], bufs[b], gsems[b])
            for b in range(NB):
                pltpu.make_async_copy(y_h.at[cbuf.at[b]], bufs[b], gsems[b]).wait()
                pltpu.async_copy(bufs[b], acc.at[rbuf.at[b]], ssems[b], add=True)
            for b in range(NB):
                jn = g * NB + b + NB

                @pl.when(jn < NCHUNK)
                def _(b=b, jn=jn):
                    pltpu.make_async_copy(bufs[b], acc.at[rbuf.at[b]], ssems[b]).wait()
                    idx_start(jn, b)

            return carry

        lax.fori_loop(0, NG, group, 0)
        for t in range(REM):
            idx_wait(t)
            pltpu.async_copy(y_h.at[cbuf.at[t]], bufs[t], gsems[t])
        for t in range(REM):
            pltpu.make_async_copy(y_h.at[cbuf.at[t]], bufs[t], gsems[t]).wait()
            pltpu.async_copy(bufs[t], acc.at[rbuf.at[t]], ssems[t], add=True)
        for j in range(max(0, NCHUNK - NB), NCHUNK):
            b = j % NB
            pltpu.make_async_copy(bufs[b], acc.at[rbuf.at[b]], ssems[b]).wait()
        plsc.subcore_barrier()
        pltpu.sync_copy(acc.at[pl.ds(r0, rows_per)],
                        out_h.at[cid, pl.ds(r0, rows_per)])

    return k(y, row_f, col_f, zfull)


def _tc_pre(degp, x, *, BR=2000):
    N, D = x.shape

    def body(da_ref, db_ref, x_ref, y_ref, s_ref, s2_ref):
        deg = da_ref[0, :, 0:1] + db_ref[0, :, 0:1] + 1.0
        s = lax.rsqrt(deg)
        y_ref[...] = x_ref[...] * s
        s_ref[...] = s
        s2_ref[...] = s * s

    return pl.pallas_call(
        body,
        grid=(N // BR,),
        in_specs=[
            pl.BlockSpec((1, BR, 128), lambda i: (0, i, 0)),
            pl.BlockSpec((1, BR, 128), lambda i: (1, i, 0)),
            pl.BlockSpec((BR, D), lambda i: (i, 0)),
        ],
        out_specs=[
            pl.BlockSpec((BR, D), lambda i: (i, 0)),
            pl.BlockSpec((BR, 1), lambda i: (i, 0)),
            pl.BlockSpec((BR, 1), lambda i: (i, 0)),
        ],
        out_shape=[
            jax.ShapeDtypeStruct((N, D), jnp.float32),
            jax.ShapeDtypeStruct((N, 1), jnp.float32),
            jax.ShapeDtypeStruct((N, 1), jnp.float32),
        ],
    )(degp, degp, x)


def _tc_post(f_col, zz, y, *, BR=2000):
    N, D = y.shape

    def body(f_ref, a_ref, b_ref, y_ref, o_ref):
        o_ref[...] = (a_ref[0] + b_ref[0] + y_ref[...]) * f_ref[...]

    return pl.pallas_call(
        body,
        grid=(N // BR,),
        in_specs=[
            pl.BlockSpec((BR, 1), lambda i: (i, 0)),
            pl.BlockSpec((1, BR, D), lambda i: (0, i, 0)),
            pl.BlockSpec((1, BR, D), lambda i: (1, i, 0)),
            pl.BlockSpec((BR, D), lambda i: (i, 0)),
        ],
        out_specs=pl.BlockSpec((BR, D), lambda i: (i, 0)),
        out_shape=jax.ShapeDtypeStruct((N, D), jnp.float32),
    )(f_col, zz, zz, y)


def kernel(x, edge_index):
    N, D = x.shape
    E = edge_index.shape[1]
    CH = 80
    EC = E // _NT
    NCHUNK = EC // CH
    NP = ((N + 16 * 16 - 1) // (16 * 16)) * (16 * 16)

    row = edge_index[0]
    col = edge_index[1]
    row3 = row.reshape(_NT, NCHUNK, CH)
    ones_b = jnp.ones((CH, 128), jnp.float32)
    zfull = jnp.zeros((NP, D), jnp.float32)

    degp = _sc_count(row3, ones_b, zfull, NP=NP, CH=CH, NCHUNK=NCHUNK)

    y, sv, s2v = _tc_pre(degp, x)
    z = _sc_spmm(y, row, col, zfull, NP=NP, D=D, CH=CH, NCHUNK=NCHUNK)
    y2 = _tc_post(s2v, z, y)
    w = _sc_spmm(y2, row, col, zfull, NP=NP, D=D, CH=CH, NCHUNK=NCHUNK)
    return _tc_post(sv, w, y2)


# speedup vs baseline: 1.0075x; 1.0075x over previous
import functools

import jax
import jax.numpy as jnp
from jax import lax
from jax.experimental import pallas as pl
from jax.experimental.pallas import tpu as pltpu
from jax.experimental.pallas import tpu_sc as plsc

_NC = 2
_NS = 16
_NT = _NC * _NS


def _mesh():
    return plsc.VectorSubcoreMesh(
        core_axis_name="c", subcore_axis_name="s",
        num_cores=_NC, num_subcores=_NS)


def _sc_count(row3, ones_b, zb, *, NP, CH, NCHUNK):
    rows_per = NP // _NS
    LAG = 8

    @functools.partial(
        pl.kernel,
        out_type=jax.ShapeDtypeStruct((_NC, NP, 128), jnp.float32),
        mesh=_mesh(),
        scratch_types=[
            pltpu.VMEM((NCHUNK, CH), jnp.int32),
            pltpu.VMEM((CH, 128), jnp.float32),
            pltpu.VMEM_SHARED((NP, 128), jnp.float32),
            pltpu.SemaphoreType.DMA,
        ],
    )
    def k(row_h, ones_h, z_h, out_h, ridx, obuf, acc, ssem):
        cid = lax.axis_index("c")
        sid = lax.axis_index("s")
        tid = cid * _NS + sid
        pltpu.sync_copy(row_h.at[tid], ridx)
        pltpu.sync_copy(ones_h, obuf)
        r0 = sid * rows_per
        pltpu.sync_copy(z_h.at[pl.ds(r0, rows_per)], acc.at[pl.ds(r0, rows_per)])
        plsc.subcore_barrier()

        def body(j, carry):
            pltpu.async_copy(obuf, acc.at[ridx.at[j]], ssem, add=True)

            @pl.when(j >= LAG)
            def _():
                pltpu.make_async_copy(obuf, acc.at[ridx.at[0]], ssem).wait()

            return carry

        lax.fori_loop(0, NCHUNK, body, 0)

        def drain(j, carry):
            pltpu.make_async_copy(obuf, acc.at[ridx.at[0]], ssem).wait()
            return carry

        lax.fori_loop(0, min(LAG, NCHUNK), drain, 0)
        plsc.subcore_barrier()
        pltpu.sync_copy(acc.at[pl.ds(r0, rows_per)],
                        out_h.at[cid, pl.ds(r0, rows_per)])

    return k(row3, ones_b, zb)


def _sc_spmm(y, row_f, col_f, zfull, *, NP, D, CH, NCHUNK):
    rows_per = NP // _NS
    NB = 4
    NG = NCHUNK // NB
    REM = NCHUNK - NG * NB

    @functools.partial(
        pl.kernel,
        out_type=jax.ShapeDtypeStruct((_NC, NP, D), jnp.float32),
        mesh=_mesh(),
        scratch_types=[
            pltpu.VMEM((NB, CH), jnp.int32),
            pltpu.VMEM((NB, CH), jnp.int32),
            *[pltpu.VMEM((CH, D), jnp.float32) for _ in range(NB)],
            pltpu.VMEM_SHARED((NP, D), jnp.float32),
            *[pltpu.SemaphoreType.DMA for _ in range(3 * NB)],
        ],
    )
    def k(y_h, row_h, col_h, z_h, out_h, rbuf, cbuf, b0, b1, b2, b3, acc,
          g0, g1, g2, g3, s0, s1, s2, s3, i0, i1, i2, i3):
        bufs = [b0, b1, b2, b3]
        gsems = [g0, g1, g2, g3]
        ssems = [s0, s1, s2, s3]
        isems = [i0, i1, i2, i3]
        cid = lax.axis_index("c")
        sid = lax.axis_index("s")
        tid = cid * _NS + sid
        r0 = sid * rows_per

        def idx_start(j, b):
            pltpu.async_copy(row_h.at[tid, j], rbuf.at[b], isems[b])
            pltpu.async_copy(col_h.at[tid, j], cbuf.at[b], isems[b])

        def idx_wait(b):
            pltpu.make_async_copy(row_h.at[0, 0], rbuf.at[b], isems[b]).wait()
            pltpu.make_async_copy(row_h.at[0, 0], cbuf.at[b], isems[b]).wait()

        for b in range(NB):
            idx_start(b, b)
        pltpu.sync_copy(z_h.at[pl.ds(r0, rows_per)], acc.at[pl.ds(r0, rows_per)])
        plsc.subcore_barrier()

        def group(g, carry):
            for b in range(NB):
                idx_wait(b)
                pltpu.async_copy(y_h.at[cbuf.at[b]], bufs[b], gsems[b])
            for b in range(NB):
                pltpu.make_async_copy(y_h.at[cbuf.at[b]], bufs[b], gsems[b]).wait()
                pltpu.async_copy(bufs[b], acc.at[rbuf.at[b]], ssems[b], add=True)
            for b in range(NB):
                jn = g * NB + b + NB

                @pl.when(jn < NCHUNK)
                def _(b=b, jn=jn):
                    pltpu.make_async_copy(bufs[b], acc.at[rbuf.at[b]], ssems[b]).wait()
                    idx_start(jn, b)

            return carry

        lax.fori_loop(0, NG, group, 0)
        for t in range(REM):
            idx_wait(t)
            pltpu.async_copy(y_h.at[cbuf.at[t]], bufs[t], gsems[t])
        for t in range(REM):
            pltpu.make_async_copy(y_h.at[cbuf.at[t]], bufs[t], gsems[t]).wait()
            pltpu.async_copy(bufs[t], acc.at[rbuf.at[t]], ssems[t], add=True)
        for j in range(max(0, NCHUNK - NB), NCHUNK):
            b = j % NB
            pltpu.make_async_copy(bufs[b], acc.at[rbuf.at[b]], ssems[b]).wait()
        plsc.subcore_barrier()
        pltpu.sync_copy(acc.at[pl.ds(r0, rows_per)],
                        out_h.at[cid, pl.ds(r0, rows_per)])

    return k(y, row_f, col_f, zfull)


def _tc_pre(degp, x, *, BR=2000):
    N, D = x.shape

    def body(da_ref, db_ref, x_ref, y_ref, s_ref, s2_ref):
        deg = da_ref[0, :, 0:1] + db_ref[0, :, 0:1] + 1.0
        s = lax.rsqrt(deg)
        y_ref[...] = x_ref[...] * s
        s_ref[...] = s
        s2_ref[...] = s * s

    return pl.pallas_call(
        body,
        grid=(N // BR,),
        in_specs=[
            pl.BlockSpec((1, BR, 128), lambda i: (0, i, 0)),
            pl.BlockSpec((1, BR, 128), lambda i: (1, i, 0)),
            pl.BlockSpec((BR, D), lambda i: (i, 0)),
        ],
        out_specs=[
            pl.BlockSpec((BR, D), lambda i: (i, 0)),
            pl.BlockSpec((BR, 1), lambda i: (i, 0)),
            pl.BlockSpec((BR, 1), lambda i: (i, 0)),
        ],
        out_shape=[
            jax.ShapeDtypeStruct((N, D), jnp.float32),
            jax.ShapeDtypeStruct((N, 1), jnp.float32),
            jax.ShapeDtypeStruct((N, 1), jnp.float32),
        ],
    )(degp, degp, x)


def _tc_post(f_col, zz, y, *, BR=2000):
    N, D = y.shape

    def body(f_ref, a_ref, b_ref, y_ref, o_ref):
        o_ref[...] = (a_ref[0] + b_ref[0] + y_ref[...]) * f_ref[...]

    return pl.pallas_call(
        body,
        grid=(N // BR,),
        in_specs=[
            pl.BlockSpec((BR, 1), lambda i: (i, 0)),
            pl.BlockSpec((1, BR, D), lambda i: (0, i, 0)),
            pl.BlockSpec((1, BR, D), lambda i: (1, i, 0)),
            pl.BlockSpec((BR, D), lambda i: (i, 0)),
        ],
        out_specs=pl.BlockSpec((BR, D), lambda i: (i, 0)),
        out_shape=jax.ShapeDtypeStruct((N, D), jnp.float32),
    )(f_col, zz, zz, y)


def kernel(x, edge_index):
    N, D = x.shape
    E = edge_index.shape[1]
    CH = 80
    EC = E // _NT
    NCHUNK = EC // CH
    NP = ((N + 16 * 16 - 1) // (16 * 16)) * (16 * 16)

    row3 = edge_index[0].reshape(_NT, NCHUNK, CH)
    col3 = edge_index[1].reshape(_NT, NCHUNK, CH)
    ones_b = jnp.ones((CH, 128), jnp.float32)
    zfull = jnp.zeros((NP, D), jnp.float32)

    degp = _sc_count(row3, ones_b, zfull, NP=NP, CH=CH, NCHUNK=NCHUNK)

    y, sv, s2v = _tc_pre(degp, x)
    z = _sc_spmm(y, row3, col3, zfull, NP=NP, D=D, CH=CH, NCHUNK=NCHUNK)
    y2 = _tc_post(s2v, z, y)
    w = _sc_spmm(y2, row3, col3, zfull, NP=NP, D=D, CH=CH, NCHUNK=NCHUNK)
    return _tc_post(sv, w, y2)


# speedup vs baseline: 1.0098x; 1.0023x over previous
import functools

import jax
import jax.numpy as jnp
from jax import lax
from jax.experimental import pallas as pl
from jax.experimental.pallas import tpu as pltpu
from jax.experimental.pallas import tpu_sc as plsc

_NC = 2
_NS = 16
_NT = _NC * _NS


def _mesh():
    return plsc.VectorSubcoreMesh(
        core_axis_name="c", subcore_axis_name="s",
        num_cores=_NC, num_subcores=_NS)


def _sc_count(row3, ones_b, zb, *, NP, CH, NCHUNK):
    rows_per = NP // _NS
    LAG = 16

    @functools.partial(
        pl.kernel,
        out_type=jax.ShapeDtypeStruct((_NC, NP, 128), jnp.float32),
        mesh=_mesh(),
        scratch_types=[
            pltpu.VMEM((NCHUNK, CH), jnp.int32),
            pltpu.VMEM((CH, 128), jnp.float32),
            pltpu.VMEM_SHARED((NP, 128), jnp.float32),
            pltpu.SemaphoreType.DMA,
        ],
    )
    def k(row_h, ones_h, z_h, out_h, ridx, obuf, acc, ssem):
        cid = lax.axis_index("c")
        sid = lax.axis_index("s")
        tid = cid * _NS + sid
        pltpu.sync_copy(row_h.at[tid], ridx)
        pltpu.sync_copy(ones_h, obuf)
        r0 = sid * rows_per
        pltpu.sync_copy(z_h.at[pl.ds(r0, rows_per)], acc.at[pl.ds(r0, rows_per)])
        plsc.subcore_barrier()

        def body(j, carry):
            pltpu.async_copy(obuf, acc.at[ridx.at[j]], ssem, add=True)

            @pl.when(j >= LAG)
            def _():
                pltpu.make_async_copy(obuf, acc.at[ridx.at[0]], ssem).wait()

            return carry

        lax.fori_loop(0, NCHUNK, body, 0)

        def drain(j, carry):
            pltpu.make_async_copy(obuf, acc.at[ridx.at[0]], ssem).wait()
            return carry

        lax.fori_loop(0, min(LAG, NCHUNK), drain, 0)
        plsc.subcore_barrier()
        pltpu.sync_copy(acc.at[pl.ds(r0, rows_per)],
                        out_h.at[cid, pl.ds(r0, rows_per)])

    return k(row3, ones_b, zb)


def _sc_spmm(y, row_f, col_f, zfull, *, NP, D, CH, NCHUNK):
    rows_per = NP // _NS
    NB = 4
    NG = NCHUNK // NB
    REM = NCHUNK - NG * NB

    @functools.partial(
        pl.kernel,
        out_type=jax.ShapeDtypeStruct((_NC, NP, D), jnp.float32),
        mesh=_mesh(),
        scratch_types=[
            pltpu.VMEM((NB, CH), jnp.int32),
            pltpu.VMEM((NB, CH), jnp.int32),
            *[pltpu.VMEM((CH, D), jnp.float32) for _ in range(NB)],
            pltpu.VMEM_SHARED((NP, D), jnp.float32),
            *[pltpu.SemaphoreType.DMA for _ in range(3 * NB)],
        ],
    )
    def k(y_h, row_h, col_h, z_h, out_h, rbuf, cbuf, b0, b1, b2, b3, acc,
          g0, g1, g2, g3, s0, s1, s2, s3, i0, i1, i2, i3):
        bufs = [b0, b1, b2, b3]
        gsems = [g0, g1, g2, g3]
        ssems = [s0, s1, s2, s3]
        isems = [i0, i1, i2, i3]
        cid = lax.axis_index("c")
        sid = lax.axis_index("s")
        tid = cid * _NS + sid
        r0 = sid * rows_per

        def idx_start(j, b):
            pltpu.async_copy(row_h.at[tid, j], rbuf.at[b], isems[b])
            pltpu.async_copy(col_h.at[tid, j], cbuf.at[b], isems[b])

        def idx_wait(b):
            pltpu.make_async_copy(row_h.at[0, 0], rbuf.at[b], isems[b]).wait()
            pltpu.make_async_copy(row_h.at[0, 0], cbuf.at[b], isems[b]).wait()

        for b in range(NB):
            idx_start(b, b)
        pltpu.sync_copy(z_h.at[pl.ds(r0, rows_per)], acc.at[pl.ds(r0, rows_per)])
        plsc.subcore_barrier()

        def group(g, carry):
            for b in range(NB):
                idx_wait(b)
                pltpu.async_copy(y_h.at[cbuf.at[b]], bufs[b], gsems[b])
            for b in range(NB):
                pltpu.make_async_copy(y_h.at[cbuf.at[b]], bufs[b], gsems[b]).wait()
                pltpu.async_copy(bufs[b], acc.at[rbuf.at[b]], ssems[b], add=True)
            for b in range(NB):
                jn = g * NB + b + NB

                @pl.when(jn < NCHUNK)
                def _(b=b, jn=jn):
                    pltpu.make_async_copy(bufs[b], acc.at[rbuf.at[b]], ssems[b]).wait()
                    idx_start(jn, b)

            return carry

        lax.fori_loop(0, NG, group, 0)
        for t in range(REM):
            idx_wait(t)
            pltpu.async_copy(y_h.at[cbuf.at[t]], bufs[t], gsems[t])
        for t in range(REM):
            pltpu.make_async_copy(y_h.at[cbuf.at[t]], bufs[t], gsems[t]).wait()
            pltpu.async_copy(bufs[t], acc.at[rbuf.at[t]], ssems[t], add=True)
        for j in range(max(0, NCHUNK - NB), NCHUNK):
            b = j % NB
            pltpu.make_async_copy(bufs[b], acc.at[rbuf.at[b]], ssems[b]).wait()
        plsc.subcore_barrier()
        pltpu.sync_copy(acc.at[pl.ds(r0, rows_per)],
                        out_h.at[cid, pl.ds(r0, rows_per)])

    return k(y, row_f, col_f, zfull)


def _tc_pre(degp, x, *, BR=2000):
    N, D = x.shape

    def body(da_ref, db_ref, x_ref, y_ref, s_ref, s2_ref):
        deg = da_ref[0, :, 0:1] + db_ref[0, :, 0:1] + 1.0
        s = lax.rsqrt(deg)
        y_ref[...] = x_ref[...] * s
        s_ref[...] = s
        s2_ref[...] = s * s

    return pl.pallas_call(
        body,
        grid=(N // BR,),
        in_specs=[
            pl.BlockSpec((1, BR, 128), lambda i: (0, i, 0)),
            pl.BlockSpec((1, BR, 128), lambda i: (1, i, 0)),
            pl.BlockSpec((BR, D), lambda i: (i, 0)),
        ],
        out_specs=[
            pl.BlockSpec((BR, D), lambda i: (i, 0)),
            pl.BlockSpec((BR, 1), lambda i: (i, 0)),
            pl.BlockSpec((BR, 1), lambda i: (i, 0)),
        ],
        out_shape=[
            jax.ShapeDtypeStruct((N, D), jnp.float32),
            jax.ShapeDtypeStruct((N, 1), jnp.float32),
            jax.ShapeDtypeStruct((N, 1), jnp.float32),
        ],
    )(degp, degp, x)


def _tc_post(f_col, zz, y, *, BR=2000):
    N, D = y.shape

    def body(f_ref, a_ref, b_ref, y_ref, o_ref):
        o_ref[...] = (a_ref[0] + b_ref[0] + y_ref[...]) * f_ref[...]

    return pl.pallas_call(
        body,
        grid=(N // BR,),
        in_specs=[
            pl.BlockSpec((BR, 1), lambda i: (i, 0)),
            pl.BlockSpec((1, BR, D), lambda i: (0, i, 0)),
            pl.BlockSpec((1, BR, D), lambda i: (1, i, 0)),
            pl.BlockSpec((BR, D), lambda i: (i, 0)),
        ],
        out_specs=pl.BlockSpec((BR, D), lambda i: (i, 0)),
        out_shape=jax.ShapeDtypeStruct((N, D), jnp.float32),
    )(f_col, zz, zz, y)


def kernel(x, edge_index):
    N, D = x.shape
    E = edge_index.shape[1]
    CH = 80
    EC = E // _NT
    NCHUNK = EC // CH
    NP = ((N + 16 * 16 - 1) // (16 * 16)) * (16 * 16)

    row3 = edge_index[0].reshape(_NT, NCHUNK, CH)
    col3 = edge_index[1].reshape(_NT, NCHUNK, CH)
    ones_b = jnp.ones((CH, 128), jnp.float32)
    zfull = jnp.zeros((NP, D), jnp.float32)

    degp = _sc_count(row3, ones_b, zfull, NP=NP, CH=CH, NCHUNK=NCHUNK)

    y, sv, s2v = _tc_pre(degp, x)
    z = _sc_spmm(y, row3, col3, zfull, NP=NP, D=D, CH=CH, NCHUNK=NCHUNK)
    y2 = _tc_post(s2v, z, y)
    w = _sc_spmm(y2, row3, col3, zfull, NP=NP, D=D, CH=CH, NCHUNK=NCHUNK)
    return _tc_post(sv, w, y2)
